# baseline (speedup 1.0000x reference)
import jax
import jax.numpy as jnp
from jax.experimental import pallas as pl


_T_BLK = 256


def _body(times_ref, w_ref, b_ref, x_ref, pos_ref, o_ref):
    ti = pl.program_id(0)
    B = times_ref.shape[0]
    tt = times_ref[pl.ds(0, B), 0, pl.ds(ti * _T_BLK, _T_BLK)]
    w = w_ref[0, :]
    bb = b_ref[0, :]
    time_emb = tt[:, :, None] * w[None, None, :] + bb[None, None, :]
    o_ref[...] = x_ref[...] + pos_ref[...][None, :, :] + time_emb


def kernel(x, times, pos_table, W_time, b_time):
    B, T, D = x.shape
    n_t = T // _T_BLK
    times3 = times.reshape(B, 1, T)
    w2 = W_time.reshape(1, D)
    b2 = b_time.reshape(1, D)

    out = pl.pallas_call(
        _body,
        grid=(n_t,),
        in_specs=[
            pl.BlockSpec((B, 1, T), lambda ti: (0, 0, 0)),
            pl.BlockSpec((1, D), lambda ti: (0, 0)),
            pl.BlockSpec((1, D), lambda ti: (0, 0)),
            pl.BlockSpec((B, _T_BLK, D), lambda ti: (0, ti, 0)),
            pl.BlockSpec((_T_BLK, D), lambda ti: (ti, 0)),
        ],
        out_specs=pl.BlockSpec((B, _T_BLK, D), lambda ti: (0, ti, 0)),
        out_shape=jax.ShapeDtypeStruct((B, T, D), x.dtype),
    )(times3, w2, b2, x, pos_table)
    return out


# speedup vs baseline: 1.0083x; 1.0083x over previous
import jax
import jax.numpy as jnp
from jax.experimental import pallas as pl


_T_BLK = 512


def _body(times_ref, w_ref, b_ref, x_ref, pos_ref, o_ref):
    ti = pl.program_id(0)
    B = times_ref.shape[0]
    tt = times_ref[pl.ds(0, B), 0, pl.ds(ti * _T_BLK, _T_BLK)]
    w = w_ref[0, :]
    bb = b_ref[0, :]
    time_emb = tt[:, :, None] * w[None, None, :] + bb[None, None, :]
    o_ref[...] = x_ref[...] + pos_ref[...][None, :, :] + time_emb


def kernel(x, times, pos_table, W_time, b_time):
    B, T, D = x.shape
    n_t = T // _T_BLK
    times3 = times.reshape(B, 1, T)
    w2 = W_time.reshape(1, D)
    b2 = b_time.reshape(1, D)

    out = pl.pallas_call(
        _body,
        grid=(n_t,),
        in_specs=[
            pl.BlockSpec((B, 1, T), lambda ti: (0, 0, 0)),
            pl.BlockSpec((1, D), lambda ti: (0, 0)),
            pl.BlockSpec((1, D), lambda ti: (0, 0)),
            pl.BlockSpec((B, _T_BLK, D), lambda ti: (0, ti, 0)),
            pl.BlockSpec((_T_BLK, D), lambda ti: (ti, 0)),
        ],
        out_specs=pl.BlockSpec((B, _T_BLK, D), lambda ti: (0, ti, 0)),
        out_shape=jax.ShapeDtypeStruct((B, T, D), x.dtype),
    )(times3, w2, b2, x, pos_table)
    return out


# speedup vs baseline: 1.0101x; 1.0018x over previous
import jax
from jax.experimental import pallas as pl


_T_BLK = 512


def _body(times_ref, w_ref, b_ref, x_ref, pos_ref, o_ref):
    ti = pl.program_id(0)
    B = times_ref.shape[0]
    tt = times_ref[pl.ds(0, B), 0, pl.ds(ti * _T_BLK, _T_BLK)]
    w = w_ref[0, :]
    bb = b_ref[0, :]
    time_emb = tt[:, :, None] * w[None, None, :] + bb[None, None, :]
    o_ref[...] = x_ref[...] + pos_ref[...][None, :, :] + time_emb


def kernel(x, times, pos_table, W_time, b_time):
    B, T, D = x.shape
    n_t = T // _T_BLK
    times3 = times.reshape(B, 1, T)
    w2 = W_time.reshape(1, D)
    b2 = b_time.reshape(1, D)

    out = pl.pallas_call(
        _body,
        grid=(n_t,),
        in_specs=[
            pl.BlockSpec((B, 1, T), lambda ti: (0, 0, 0)),
            pl.BlockSpec((1, D), lambda ti: (0, 0)),
            pl.BlockSpec((1, D), lambda ti: (0, 0)),
            pl.BlockSpec((B, _T_BLK, D), lambda ti: (0, ti, 0)),
            pl.BlockSpec((_T_BLK, D), lambda ti: (ti, 0)),
        ],
        out_specs=pl.BlockSpec((B, _T_BLK, D), lambda ti: (0, ti, 0)),
        out_shape=jax.ShapeDtypeStruct((B, T, D), x.dtype),
    )(times3, w2, b2, x, pos_table)
    return out
